# baseline (speedup 1.0000x reference)
import functools

import jax
import jax.numpy as jnp
from jax.experimental import pallas as pl
from jax.experimental.pallas import tpu as pltpu

_VMEM_LIMIT = 64 * 1024 * 1024
_BF = jnp.bfloat16
_P = 896
_PB = 768


def _pilonet_kernel(x_ref, w0, b0, w1, b1, w2, b2, w3, b3, w4, b4,
                    wf0, bf0, wf1, bf1, wf2, bf2, o_ref,
                    p0, h0, p1, h1, p2, h2, p3, h3, p4, rhs, *, B):
    f32 = jnp.float32

    for p in range(31):
        for e in range(4):
            k_lo = min(5, 8 - 2 * e)
            for kh in range(5):
                p0[15 * kh:15 * kh + 3 * k_lo, e * _P:e * _P + 25 * B] = \
                    x_ref[2 * p + kh, 6 * e:6 * e + 3 * k_lo, :25 * B]
                if k_lo < 5:
                    p0[15 * kh + 3 * k_lo:15 * kh + 15,
                       e * _P:e * _P + 24 * B] = \
                        x_ref[2 * p + kh, 0:3 * (5 - k_lo), B:25 * B]
        for e in range(4):
            acc = jnp.dot(w0[...], p0[:, e * _P:(e + 1) * _P],
                          preferred_element_type=f32)
            h0[p, e] = jnp.maximum(acc + b0[...], 0.0).astype(_BF)

    for p in range(14):
        for e in range(2):
            for kw in range(5):
                t = 2 * e + kw
                p1[kw, :, :, e * _PB:(e + 1) * _PB] = \
                    h0[2 * p:2 * p + 5, t % 4, :, (t // 4) * B:(t // 4 + 24) * B]
        for e in range(2):
            acc = jnp.dot(w1[...], p1[:, :, :, e * _PB:(e + 1) * _PB]
                          .reshape(600, _PB), preferred_element_type=f32)
            h1[p, e] = jnp.maximum(acc + b1[...], 0.0).astype(_BF)

    for q in range(5):
        for kw in range(5):
            p2[kw, :, :, q * _PB:q * _PB + 22 * B] = \
                h1[2 * q:2 * q + 5, kw % 2, :, (kw // 2) * B:(kw // 2 + 22) * B]
    for q in range(5):
        acc = jnp.dot(w2[...], p2[:, :, :, q * _PB:(q + 1) * _PB]
                      .reshape(1000, _PB), preferred_element_type=f32)
        h2[q] = jnp.maximum(acc + b2[...], 0.0).astype(_BF)

    for q in range(3):
        for kw in range(3):
            p3[kw, :, :, q * _PB:q * _PB + 20 * B] = \
                h2[q:q + 3, :, kw * B:(kw + 20) * B]
    for q in range(3):
        acc = jnp.dot(w3[...], p3[:, :, :, q * _PB:(q + 1) * _PB]
                      .reshape(432, _PB), preferred_element_type=f32)
        h3[q] = jnp.maximum(acc + b3[...], 0.0).astype(_BF)

    for kw in range(3):
        p4[kw, :, :, :18 * B] = h3[:, :, kw * B:(kw + 18) * B]
    acc = jnp.dot(w4[...], p4[...].reshape(576, _PB),
                  preferred_element_type=f32)
    acc = jnp.maximum(acc + b4[...], 0.0)

    for w in range(18):
        rhs[w] = acc[:, w * B:(w + 1) * B].astype(_BF)
    y = jnp.dot(wf0[...], rhs[...].reshape(1152, B), preferred_element_type=f32)
    y = jnp.maximum(y + bf0[...], 0.0)
    y = jnp.dot(wf1[...], y.astype(_BF), preferred_element_type=f32)
    y = jnp.maximum(y + bf1[...], 0.0)
    y = jnp.dot(wf2[...], y.astype(_BF), preferred_element_type=f32)
    o_ref[...] = y + bf2[...]


def _reorder_taps(w, k, ic):
    oc = w.shape[0]
    return (w.reshape(oc, k, k, ic).transpose(0, 2, 1, 3)
             .reshape(oc, k * k * ic).astype(_BF))


def kernel(w0, b0, w1, b1, w2, b2, w3, b3, w4, b4,
           wf0, bf0, wf1, bf1, wf2, bf2, x):
    n, c, hh, ww = x.shape
    assert (c, hh, ww) == (3, 66, 200), x.shape
    B = 32
    g = -(-n // B)
    n_pad = g * B
    ncls = wf2.shape[0]

    xp = jnp.pad(x, ((0, n_pad - n), (0, 0), (0, 0), (0, 0))).astype(_BF)
    xt = jnp.transpose(xp, (1, 2, 3, 0))
    xt = xt.reshape(3, 66, 25, 8, g, B)
    xs = jnp.transpose(xt, (4, 1, 3, 0, 2, 5)).reshape(g, 66, 24, 25 * B)

    w0r = w0.reshape(24, 5, 5, 8)[:, :, :, :3].reshape(24, 75).astype(_BF)
    w1r = _reorder_taps(w1, 5, 24)
    w2r = _reorder_taps(w2, 5, 40)
    w3r = _reorder_taps(w3, 3, 48)
    w4r = _reorder_taps(w4, 3, 64)

    _body = functools.partial(_pilonet_kernel, B=B)
    wspec = lambda a: pl.BlockSpec(a.shape, lambda i: (0,) * a.ndim)
    args = (w0r, b0, w1r, b1, w2r, b2, w3r, b3, w4r, b4,
            wf0.astype(_BF), bf0, wf1.astype(_BF), bf1, wf2.astype(_BF), bf2)
    out = pl.pallas_call(
        _body,
        out_shape=jax.ShapeDtypeStruct((g, ncls, B), jnp.float32),
        grid=(g,),
        in_specs=[pl.BlockSpec((None, 66, 24, 25 * B),
                               lambda i: (i, 0, 0, 0))]
                 + [wspec(a) for a in args],
        out_specs=pl.BlockSpec((None, ncls, B), lambda i: (i, 0, 0)),
        scratch_shapes=[
            pltpu.VMEM((75, 4 * _P), _BF),
            pltpu.VMEM((31, 4, 24, _P), _BF),
            pltpu.VMEM((5, 5, 24, 2 * _PB), _BF),
            pltpu.VMEM((14, 2, 40, _PB), _BF),
            pltpu.VMEM((5, 5, 40, 5 * _PB), _BF),
            pltpu.VMEM((5, 48, _PB), _BF),
            pltpu.VMEM((3, 3, 48, 3 * _PB), _BF),
            pltpu.VMEM((3, 64, _PB), _BF),
            pltpu.VMEM((3, 3, 64, _PB), _BF),
            pltpu.VMEM((18, 64, B), _BF),
        ],
        compiler_params=pltpu.CompilerParams(
            dimension_semantics=("parallel",),
            vmem_limit_bytes=_VMEM_LIMIT),
    )(xs, *args)

    return jnp.transpose(out, (0, 2, 1)).reshape(n_pad, ncls)[:n]


# speedup vs baseline: 1.0200x; 1.0200x over previous
import functools

import jax
import jax.numpy as jnp
from jax.experimental import pallas as pl
from jax.experimental.pallas import tpu as pltpu

_VMEM_LIMIT = 64 * 1024 * 1024
_BF = jnp.bfloat16
_P = 896
_PB = 768


def _pilonet_kernel(x_ref, w0, b0, w1, b1, w2, b2, w3, b3, w4, b4,
                    wf0, bf0, wf1, bf1, wf2, bf2, o_ref,
                    p0, h0, p1, h1, p2, h2, p3, h3, p4, rhs, *, B):
    f32 = jnp.float32

    for p in range(31):
        for e in range(4):
            k_lo = min(5, 8 - 2 * e)
            for kh in range(5):
                p0[15 * kh:15 * kh + 3 * k_lo, e * _P:e * _P + 25 * B] = \
                    x_ref[2 * p + kh, 6 * e:6 * e + 3 * k_lo, :25 * B]
                if k_lo < 5:
                    p0[15 * kh + 3 * k_lo:15 * kh + 15,
                       e * _P:e * _P + 24 * B] = \
                        x_ref[2 * p + kh, 0:3 * (5 - k_lo), B:25 * B]
        acc = jnp.dot(w0[...], p0[...].reshape(75, 4 * _P),
                      preferred_element_type=f32)
        acc = jnp.maximum(acc + b0[...], 0.0)
        for e in range(4):
            h0[p, e] = acc[:, e * _P:(e + 1) * _P].astype(_BF)

    for p in range(14):
        for e in range(2):
            for kw in range(5):
                t = 2 * e + kw
                p1[kw, :, :, e * _PB:(e + 1) * _PB] = \
                    h0[2 * p:2 * p + 5, t % 4, :, (t // 4) * B:(t // 4 + 24) * B]
        acc = jnp.dot(w1[...], p1[...].reshape(600, 2 * _PB),
                      preferred_element_type=f32)
        acc = jnp.maximum(acc + b1[...], 0.0)
        h1[p, 0] = acc[:, :_PB].astype(_BF)
        h1[p, 1] = acc[:, _PB:].astype(_BF)

    for q in range(5):
        for kw in range(5):
            p2[kw, :, :, q * _PB:q * _PB + 22 * B] = \
                h1[2 * q:2 * q + 5, kw % 2, :, (kw // 2) * B:(kw // 2 + 22) * B]
    acc = jnp.dot(w2[...], p2[...].reshape(1000, 5 * _PB),
                  preferred_element_type=f32)
    acc = jnp.maximum(acc + b2[...], 0.0)
    for q in range(5):
        h2[q] = acc[:, q * _PB:(q + 1) * _PB].astype(_BF)

    for q in range(3):
        for kw in range(3):
            p3[kw, :, :, q * _PB:q * _PB + 20 * B] = \
                h2[q:q + 3, :, kw * B:(kw + 20) * B]
    acc = jnp.dot(w3[...], p3[...].reshape(432, 3 * _PB),
                  preferred_element_type=f32)
    acc = jnp.maximum(acc + b3[...], 0.0)
    for q in range(3):
        h3[q] = acc[:, q * _PB:(q + 1) * _PB].astype(_BF)

    for kw in range(3):
        p4[kw, :, :, :18 * B] = h3[:, :, kw * B:(kw + 18) * B]
    acc = jnp.dot(w4[...], p4[...].reshape(576, _PB),
                  preferred_element_type=f32)
    acc = jnp.maximum(acc + b4[...], 0.0)

    for w in range(18):
        rhs[w] = acc[:, w * B:(w + 1) * B].astype(_BF)
    y = jnp.dot(wf0[...], rhs[...].reshape(1152, B), preferred_element_type=f32)
    y = jnp.maximum(y + bf0[...], 0.0)
    y = jnp.dot(wf1[...], y.astype(_BF), preferred_element_type=f32)
    y = jnp.maximum(y + bf1[...], 0.0)
    y = jnp.dot(wf2[...], y.astype(_BF), preferred_element_type=f32)
    o_ref[...] = y + bf2[...]


def _reorder_taps(w, k, ic):
    oc = w.shape[0]
    return (w.reshape(oc, k, k, ic).transpose(0, 2, 1, 3)
             .reshape(oc, k * k * ic).astype(_BF))


def kernel(w0, b0, w1, b1, w2, b2, w3, b3, w4, b4,
           wf0, bf0, wf1, bf1, wf2, bf2, x):
    n, c, hh, ww = x.shape
    assert (c, hh, ww) == (3, 66, 200), x.shape
    B = 32
    g = -(-n // B)
    n_pad = g * B
    ncls = wf2.shape[0]

    xp = jnp.pad(x, ((0, n_pad - n), (0, 0), (0, 0), (0, 0))).astype(_BF)
    xt = jnp.transpose(xp, (1, 2, 3, 0))
    xt = xt.reshape(3, 66, 25, 8, g, B)
    xs = jnp.transpose(xt, (4, 1, 3, 0, 2, 5)).reshape(g, 66, 24, 25 * B)

    w0r = w0.reshape(24, 5, 5, 8)[:, :, :, :3].reshape(24, 75).astype(_BF)
    w1r = _reorder_taps(w1, 5, 24)
    w2r = _reorder_taps(w2, 5, 40)
    w3r = _reorder_taps(w3, 3, 48)
    w4r = _reorder_taps(w4, 3, 64)

    _body = functools.partial(_pilonet_kernel, B=B)
    wspec = lambda a: pl.BlockSpec(a.shape, lambda i: (0,) * a.ndim)
    args = (w0r, b0, w1r, b1, w2r, b2, w3r, b3, w4r, b4,
            wf0.astype(_BF), bf0, wf1.astype(_BF), bf1, wf2.astype(_BF), bf2)
    out = pl.pallas_call(
        _body,
        out_shape=jax.ShapeDtypeStruct((g, ncls, B), jnp.float32),
        grid=(g,),
        in_specs=[pl.BlockSpec((None, 66, 24, 25 * B),
                               lambda i: (i, 0, 0, 0))]
                 + [wspec(a) for a in args],
        out_specs=pl.BlockSpec((None, ncls, B), lambda i: (i, 0, 0)),
        scratch_shapes=[
            pltpu.VMEM((75, 4 * _P), _BF),
            pltpu.VMEM((31, 4, 24, _P), _BF),
            pltpu.VMEM((5, 5, 24, 2 * _PB), _BF),
            pltpu.VMEM((14, 2, 40, _PB), _BF),
            pltpu.VMEM((5, 5, 40, 5 * _PB), _BF),
            pltpu.VMEM((5, 48, _PB), _BF),
            pltpu.VMEM((3, 3, 48, 3 * _PB), _BF),
            pltpu.VMEM((3, 64, _PB), _BF),
            pltpu.VMEM((3, 3, 64, _PB), _BF),
            pltpu.VMEM((18, 64, B), _BF),
        ],
        compiler_params=pltpu.CompilerParams(
            dimension_semantics=("parallel",),
            vmem_limit_bytes=_VMEM_LIMIT),
    )(xs, *args)

    return jnp.transpose(out, (0, 2, 1)).reshape(n_pad, ncls)[:n]
